# baseline (speedup 1.0000x reference)
import functools

import numpy as np
import jax
import jax.numpy as jnp
from jax import lax
from jax.experimental import pallas as pl
from jax.experimental.pallas import tpu as pltpu
from jax.experimental.pallas import tpu_sc as plsc

DIM = 64
NW = 32
LANE = 128
TPC = 256


def _gather3(t0, t1, t2, idxT, n_tok):
    tpw = n_tok // NW
    cpw = tpw // TPC
    mesh = plsc.VectorSubcoreMesh(core_axis_name="c", subcore_axis_name="s")

    @functools.partial(
        pl.kernel,
        mesh=mesh,
        compiler_params=pltpu.CompilerParams(use_tc_tiling_on_sc=False),
        out_type=jax.ShapeDtypeStruct((n_tok, DIM), jnp.float32),
        scratch_types=[
            pltpu.VMEM((3, tpw), jnp.int32),
            pltpu.VMEM((3, TPC, DIM), jnp.float32),
            pltpu.VMEM((TPC, DIM), jnp.float32),
            pltpu.SemaphoreType.DMA,
        ],
    )
    def body(t0_h, t1_h, t2_h, idx_h, out_h, idx_v, g_v, s_v, sem):
        wid = lax.axis_index("s") * 2 + lax.axis_index("c")
        for t in range(3):
            pltpu.sync_copy(idx_h.at[t, pl.ds(wid * tpw, tpw)], idx_v.at[t])

        def chunk(ch, c):
            cps = []
            for t, tbl in enumerate((t0_h, t1_h, t2_h)):
                for j in range(TPC // LANE):
                    cps.append(
                        pltpu.async_copy(
                            tbl.at[idx_v.at[t, pl.ds(ch * TPC + j * LANE, LANE)]],
                            g_v.at[t, pl.ds(j * LANE, LANE)],
                            sem,
                        )
                    )
            for cp_ in cps:
                cp_.wait()

            def tok(k, c2):
                for s in range(DIM // 16):
                    sl = pl.ds(s * 16, 16)
                    s_v[k, sl] = g_v[0, k, sl] + g_v[1, k, sl] + g_v[2, k, sl]
                return c2

            lax.fori_loop(0, TPC, tok, 0)
            pltpu.sync_copy(s_v, out_h.at[pl.ds((wid * cpw + ch) * TPC, TPC)])
            return c

        lax.fori_loop(0, cpw, chunk, 0)

    return body(t0, t1, t2, idxT)


SEQ_BLK = 8


def _pe_body(cpt_ref, l0_ref, w_ref, b_ref, xpe_ref, msk_ref):
    t = cpt_ref[...] * np.float32(np.pi)
    s1 = jnp.sin(t)
    c1 = jnp.cos(t)
    two_c1 = c1 + c1
    S = [s1, two_c1 * s1]
    C = [c1, two_c1 * c1 - 1.0]
    for _ in range(14):
        S.append(two_c1 * S[-1] - S[-2])
        C.append(two_c1 * C[-1] - C[-2])
    n_bat = cpt_ref.shape[1]
    w = w_ref[...]
    bb = b_ref[...]
    for s8 in range(SEQ_BLK):
        a = jnp.concatenate(
            [v[8 * s8:8 * s8 + 8] for v in S + C], axis=0
        )
        xpe = lax.dot_general(
            a, w, (((0,), (0,)), ((), ())),
            preferred_element_type=jnp.float32,
        )
        r = pl.ds(s8 * n_bat, n_bat)
        xpe_ref[r, :] = (xpe + bb) * 0.5
    msk_ref[...] = (l0_ref[...] < 0).astype(jnp.int32)


def _pe(cpX, l0, w2, b, n_seq, n_bat):
    grid = (n_seq // SEQ_BLK,)
    return pl.pallas_call(
        _pe_body,
        grid=grid,
        in_specs=[
            pl.BlockSpec((8 * SEQ_BLK, n_bat), lambda i: (i, 0)),
            pl.BlockSpec((SEQ_BLK, n_bat), lambda i: (i, 0)),
            pl.BlockSpec((256, DIM), lambda i: (0, 0)),
            pl.BlockSpec((1, DIM), lambda i: (0, 0)),
        ],
        out_specs=[
            pl.BlockSpec((SEQ_BLK * n_bat, DIM), lambda i: (i, 0)),
            pl.BlockSpec((SEQ_BLK, n_bat), lambda i: (i, 0)),
        ],
        out_shape=[
            jax.ShapeDtypeStruct((n_seq * n_bat, DIM), jnp.float32),
            jax.ShapeDtypeStruct((n_seq, n_bat), jnp.int32),
        ],
    )(cpX, l0, w2, b.reshape(1, DIM))


def _add_body(g_ref, xpe_ref, x_ref):
    x_ref[...] = g_ref[...] * 0.5 + xpe_ref[...]


def _add(g, xpeh, n_rows):
    r_blk = 8192
    grid = (n_rows // r_blk,)
    return pl.pallas_call(
        _add_body,
        grid=grid,
        in_specs=[
            pl.BlockSpec((r_blk, DIM), lambda i: (i, 0)),
            pl.BlockSpec((r_blk, DIM), lambda i: (i, 0)),
        ],
        out_specs=pl.BlockSpec((r_blk, DIM), lambda i: (i, 0)),
        out_shape=jax.ShapeDtypeStruct((n_rows, DIM), jnp.float32),
    )(g, xpeh)


def kernel(labels, control_points, stroke_table, startpoint_table, endpoint_table, W, b):
    b_, s_ = labels.shape[0], labels.shape[1]
    n = b_ * s_

    idxT = jnp.transpose(labels, (2, 1, 0)).reshape(3, n)
    cpX = jnp.transpose(control_points, (1, 2, 3, 0)).reshape(s_ * 8, b_)
    l0 = jnp.transpose(labels[:, :, 0], (1, 0))

    g = _gather3(stroke_table, startpoint_table, endpoint_table, idxT, n)

    w2 = W.reshape(8, 32, DIM).transpose(1, 0, 2).reshape(256, DIM)

    xpeh, msk = _pe(cpX, l0, w2, b, s_, b_)
    x = _add(g, xpeh, n)
    x = jnp.transpose(x.reshape(s_, b_, DIM), (1, 0, 2))
    return x, jnp.transpose(msk, (1, 0)).astype(jnp.bool_)


# speedup vs baseline: 1.0432x; 1.0432x over previous
import functools

import numpy as np
import jax
import jax.numpy as jnp
from jax import lax
from jax.experimental import pallas as pl
from jax.experimental.pallas import tpu as pltpu
from jax.experimental.pallas import tpu_sc as plsc

DIM = 64
NW = 32
LANE = 128
TPC = 256


def _gather3(t0, t1, t2, idxT, n_tok):
    tpw = n_tok // NW
    cpw = tpw // TPC
    mesh = plsc.VectorSubcoreMesh(core_axis_name="c", subcore_axis_name="s")

    @functools.partial(
        pl.kernel,
        mesh=mesh,
        compiler_params=pltpu.CompilerParams(use_tc_tiling_on_sc=False),
        out_type=jax.ShapeDtypeStruct((n_tok, DIM), jnp.float32),
        scratch_types=[
            pltpu.VMEM((3, tpw), jnp.int32),
            pltpu.VMEM((3, TPC, DIM), jnp.float32),
            pltpu.VMEM((TPC, DIM), jnp.float32),
            pltpu.SemaphoreType.DMA,
        ],
    )
    def body(t0_h, t1_h, t2_h, idx_h, out_h, idx_v, g_v, s_v, sem):
        wid = lax.axis_index("s") * 2 + lax.axis_index("c")
        for t in range(3):
            pltpu.sync_copy(idx_h.at[t, pl.ds(wid * tpw, tpw)], idx_v.at[t])

        def chunk(ch, c):
            cps = []
            for t, tbl in enumerate((t0_h, t1_h, t2_h)):
                for j in range(TPC // LANE):
                    cps.append(
                        pltpu.async_copy(
                            tbl.at[idx_v.at[t, pl.ds(ch * TPC + j * LANE, LANE)]],
                            g_v.at[t, pl.ds(j * LANE, LANE)],
                            sem,
                        )
                    )
            for cp_ in cps:
                cp_.wait()

            def tok(k, c2):
                for s in range(DIM // 16):
                    sl = pl.ds(s * 16, 16)
                    s_v[k, sl] = g_v[0, k, sl] + g_v[1, k, sl] + g_v[2, k, sl]
                return c2

            lax.fori_loop(0, TPC, tok, 0)
            pltpu.sync_copy(s_v, out_h.at[pl.ds((wid * cpw + ch) * TPC, TPC)])
            return c

        lax.fori_loop(0, cpw, chunk, 0)

    return body(t0, t1, t2, idxT)


SEQ_BLK = 8


def _mixer_body(g_ref, cpt_ref, l0_ref, w_ref, b_ref, x_ref, msk_ref):
    t = cpt_ref[...] * np.float32(np.pi)
    s1 = jnp.sin(t)
    c1 = jnp.cos(t)
    two_c1 = c1 + c1
    S = [s1, two_c1 * s1]
    C = [c1, two_c1 * c1 - 1.0]
    for _ in range(14):
        S.append(two_c1 * S[-1] - S[-2])
        C.append(two_c1 * C[-1] - C[-2])
    n_bat = cpt_ref.shape[1]
    w = w_ref[...]
    bb = b_ref[...]
    for s8 in range(SEQ_BLK):
        a = jnp.concatenate(
            [v[8 * s8:8 * s8 + 8] for v in S + C], axis=0
        )
        xpe = lax.dot_general(
            a, w, (((0,), (0,)), ((), ())),
            preferred_element_type=jnp.float32,
        )
        r = pl.ds(s8 * n_bat, n_bat)
        x_ref[r, :] = (g_ref[r, :] + xpe + bb) * 0.5
    msk_ref[...] = (l0_ref[...] < 0).astype(jnp.int32)


def _mixer(g, cpX, l0, w2, b, n_seq, n_bat):
    grid = (n_seq // SEQ_BLK,)
    return pl.pallas_call(
        _mixer_body,
        grid=grid,
        in_specs=[
            pl.BlockSpec((SEQ_BLK * n_bat, DIM), lambda i: (i, 0)),
            pl.BlockSpec((8 * SEQ_BLK, n_bat), lambda i: (i, 0)),
            pl.BlockSpec((SEQ_BLK, n_bat), lambda i: (i, 0)),
            pl.BlockSpec((256, DIM), lambda i: (0, 0)),
            pl.BlockSpec((1, DIM), lambda i: (0, 0)),
        ],
        out_specs=[
            pl.BlockSpec((SEQ_BLK * n_bat, DIM), lambda i: (i, 0)),
            pl.BlockSpec((SEQ_BLK, n_bat), lambda i: (i, 0)),
        ],
        out_shape=[
            jax.ShapeDtypeStruct((n_seq * n_bat, DIM), jnp.float32),
            jax.ShapeDtypeStruct((n_seq, n_bat), jnp.int32),
        ],
    )(g, cpX, l0, w2, b.reshape(1, DIM))


def kernel(labels, control_points, stroke_table, startpoint_table, endpoint_table, W, b):
    b_, s_ = labels.shape[0], labels.shape[1]
    n = b_ * s_

    idxT = jnp.transpose(labels, (2, 1, 0)).reshape(3, n)
    cpX = jnp.transpose(control_points, (1, 2, 3, 0)).reshape(s_ * 8, b_)
    l0 = jnp.transpose(labels[:, :, 0], (1, 0))

    g = _gather3(stroke_table, startpoint_table, endpoint_table, idxT, n)

    w2 = W.reshape(8, 32, DIM).transpose(1, 0, 2).reshape(256, DIM)

    x, msk = _mixer(g, cpX, l0, w2, b, s_, b_)
    x = jnp.transpose(x.reshape(s_, b_, DIM), (1, 0, 2))
    return x, jnp.transpose(msk, (1, 0)).astype(jnp.bool_)
